# baseline (speedup 1.0000x reference)
import functools

import jax
import jax.numpy as jnp
from jax import lax
from jax.experimental import pallas as pl
from jax.experimental.pallas import tpu as pltpu
from jax.experimental.pallas import tpu_sc as plsc

_B = 4096
_D = 64
_NC = 2
_NS = 16
_NW = _NC * _NS
_BPW = _B // _NW

@functools.cache
def _make_gather3():
    mesh = plsc.VectorSubcoreMesh(core_axis_name="c", subcore_axis_name="s")

    @functools.partial(
        pl.kernel,
        mesh=mesh,
        out_type=jax.ShapeDtypeStruct((3, _B, _D), jnp.float32),
        compiler_params=pltpu.CompilerParams(use_tc_tiling_on_sc=False),
        scratch_types=[
            pltpu.VMEM((_BPW,), jnp.int32),
            pltpu.VMEM((_BPW, _D), jnp.float32),
            pltpu.SemaphoreType.DMA,
        ],
    )
    def _gather3(d_tab, u_tab, doc_u, pos_v, neg_v, out, idx_v, rows_v, sem):
        wid = lax.axis_index("s") * _NC + lax.axis_index("c")
        base = wid * _BPW
        for t, (tab, idx) in enumerate(
            ((d_tab, doc_u), (u_tab, pos_v), (u_tab, neg_v))
        ):
            pltpu.sync_copy(idx.at[pl.ds(base, _BPW)], idx_v)
            pltpu.async_copy(tab.at[idx_v], rows_v, sem).wait()
            pltpu.sync_copy(rows_v, out.at[t, pl.ds(base, _BPW)])

    return _gather3


_BM = 512
_GRID = _B // _BM


def _loss_body(d_ref, v_ref, n_ref, out_ref):
    i = pl.program_id(0)
    d = d_ref[0]
    v = v_ref[0]
    n = n_ref[0]
    dn = (((1,), (1,)), ((), ()))
    s_pos = lax.dot_general(d, v, dn, preferred_element_type=jnp.float32)
    s_neg = lax.dot_general(d, n, dn, preferred_element_type=jnp.float32)

    def neg_logsig_sum(x):
        return jnp.sum(jnp.log1p(jnp.exp(-jnp.abs(x))) - jnp.minimum(x, 0.0))

    part = neg_logsig_sum(s_pos) + neg_logsig_sum(-s_neg)

    @pl.when(i == 0)
    def _init():
        out_ref[0, 0] = 0.0

    out_ref[0, 0] += part


def _loss(g):
    return pl.pallas_call(
        _loss_body,
        grid=(_GRID,),
        in_specs=[
            pl.BlockSpec((1, _BM, _D), lambda i: (0, i, 0)),
            pl.BlockSpec((1, _B, _D), lambda i: (1, 0, 0)),
            pl.BlockSpec((1, _B, _D), lambda i: (2, 0, 0)),
        ],
        out_specs=pl.BlockSpec((1, 1), lambda i: (0, 0), memory_space=pltpu.SMEM),
        out_shape=jax.ShapeDtypeStruct((1, 1), jnp.float32),
    )(g, g, g)


def kernel(doc_u, pos_v, neg_v, D_emb, U_emb, V_emb):
    g = _make_gather3()(
        D_emb,
        U_emb,
        doc_u.astype(jnp.int32),
        pos_v.astype(jnp.int32),
        neg_v.astype(jnp.int32),
    )
    return _loss(g)[0, 0]


# speedup vs baseline: 1.1194x; 1.1194x over previous
import functools

import jax
import jax.numpy as jnp
from jax import lax
from jax.experimental import pallas as pl
from jax.experimental.pallas import tpu as pltpu
from jax.experimental.pallas import tpu_sc as plsc

_B = 4096
_D = 64
_NC = 2
_NS = 16
_NW = _NC * _NS
_BPW = _B // _NW

@functools.cache
def _make_gather3():
    mesh = plsc.VectorSubcoreMesh(core_axis_name="c", subcore_axis_name="s")

    @functools.partial(
        pl.kernel,
        mesh=mesh,
        out_type=jax.ShapeDtypeStruct((3, _B, _D), jnp.float32),
        compiler_params=pltpu.CompilerParams(needs_layout_passes=False),
        scratch_types=[
            pltpu.VMEM((_BPW,), jnp.int32),
            pltpu.SemaphoreType.DMA,
        ],
    )
    def _gather3(d_tab, u_tab, doc_u, pos_v, neg_v, out, idx_v, sem):
        wid = lax.axis_index("s") * _NC + lax.axis_index("c")
        base = wid * _BPW
        lanes = lax.iota(jnp.int32, 16)
        for t, (tab, idx) in enumerate(
            ((d_tab, doc_u), (u_tab, pos_v), (u_tab, neg_v))
        ):
            pltpu.sync_copy(idx.at[pl.ds(base, _BPW)], idx_v)
            for c in range(_BPW // 16):
                vec = idx_v[pl.ds(c * 16, 16)]
                copies = []
                for r in range(16):
                    s = jnp.sum(jnp.where(lanes == r, vec, 0))
                    copies.append(
                        pltpu.async_copy(
                            tab.at[pl.ds(s, 1)],
                            out.at[t, pl.ds(base + c * 16 + r, 1)],
                            sem,
                        )
                    )
                for cp in copies:
                    cp.wait()

    return _gather3


_BM = 512
_GRID = _B // _BM


def _loss_body(d_ref, v_ref, n_ref, out_ref):
    i = pl.program_id(0)
    d = d_ref[0]
    v = v_ref[0]
    n = n_ref[0]
    dn = (((1,), (1,)), ((), ()))
    s_pos = lax.dot_general(d, v, dn, preferred_element_type=jnp.float32)
    s_neg = lax.dot_general(d, n, dn, preferred_element_type=jnp.float32)

    def neg_logsig_sum(x):
        return jnp.sum(jnp.log1p(jnp.exp(-jnp.abs(x))) - jnp.minimum(x, 0.0))

    part = neg_logsig_sum(s_pos) + neg_logsig_sum(-s_neg)

    @pl.when(i == 0)
    def _init():
        out_ref[0, 0] = 0.0

    out_ref[0, 0] += part


def _loss(g):
    return pl.pallas_call(
        _loss_body,
        grid=(_GRID,),
        in_specs=[
            pl.BlockSpec((1, _BM, _D), lambda i: (0, i, 0)),
            pl.BlockSpec((1, _B, _D), lambda i: (1, 0, 0)),
            pl.BlockSpec((1, _B, _D), lambda i: (2, 0, 0)),
        ],
        out_specs=pl.BlockSpec((1, 1), lambda i: (0, 0), memory_space=pltpu.SMEM),
        out_shape=jax.ShapeDtypeStruct((1, 1), jnp.float32),
    )(g, g, g)


def kernel(doc_u, pos_v, neg_v, D_emb, U_emb, V_emb):
    g = _make_gather3()(
        D_emb,
        U_emb,
        doc_u.astype(jnp.int32),
        pos_v.astype(jnp.int32),
        neg_v.astype(jnp.int32),
    )
    return _loss(g)[0, 0]
